# baseline (speedup 1.0000x reference)
import functools

import jax
import jax.numpy as jnp
from jax import lax
from jax.experimental import pallas as pl
from jax.experimental.pallas import tpu as pltpu
from jax.experimental.pallas import tpu_sc as plsc

N_FIELDS = 26
VOCAB = 100000
EMBED = 32
BATCH = 16384

_info = plsc.get_sparse_core_info()
_NC, _NS = _info.num_cores, _info.num_subcores
_NW = _NC * _NS
_BPW = BATCH // _NW

_VBLK = 51200
_VPAD = ((VOCAB + 1 + _VBLK - 1) // _VBLK) * _VBLK

_NF1 = N_FIELDS // 2
_NF2 = N_FIELDS - _NF1


def _repack_block(x_ref, o_ref):
    eye = jnp.eye(128, dtype=jnp.float32)
    x = x_ref[0]
    for s in range(_VBLK // 512):
        pieces = [
            jax.lax.dot_general(
                eye,
                x[:, s * 512 + d * 128:s * 512 + (d + 1) * 128],
                (((1,), (1,)), ((), ())),
                preferred_element_type=jnp.float32,
            )
            for d in range(4)
        ]
        o_ref[0, pl.ds(s * 128, 128)] = jnp.concatenate(pieces, axis=1)


def _make_repack(nf, off):
    return pl.pallas_call(
        _repack_block,
        grid=(nf, _VPAD // _VBLK),
        in_specs=[pl.BlockSpec((1, EMBED, _VBLK), lambda i, j: (i + off, 0, j))],
        out_specs=pl.BlockSpec((1, _VBLK // 4, 4 * EMBED), lambda i, j: (i, j, 0)),
        out_shape=jax.ShapeDtypeStruct((nf, _VPAD // 4, 4 * EMBED), jnp.float32),
    )


_UROWS = 4096
_UB = _UROWS * 4


def _unpack_block(y_ref, o_ref):
    r_io = lax.broadcasted_iota(jnp.int32, (128, 512), 0)
    c_io = lax.broadcasted_iota(jnp.int32, (128, 512), 1)
    sels = [(c_io == 4 * r_io + d).astype(jnp.float32) for d in range(4)]
    segs = []
    for g in range(_UROWS // 128):
        acc = None
        for d in range(4):
            piece = jax.lax.dot_general(
                y_ref[0, pl.ds(g * 128, 128), pl.ds(d * EMBED, EMBED)],
                sels[d],
                (((0,), (0,)), ((), ())),
                preferred_element_type=jnp.float32,
            )
            acc = piece if acc is None else acc + piece
        segs.append(acc)
    o_ref[0] = jnp.concatenate(segs, axis=1)


def _make_unpack(nf):
    return pl.pallas_call(
        _unpack_block,
        grid=(nf, BATCH // _UB),
        in_specs=[pl.BlockSpec((1, _UROWS, 128), lambda i, j: (i, j, 0))],
        out_specs=pl.BlockSpec((1, EMBED, _UB), lambda i, j: (i, 0, j)),
        out_shape=jax.ShapeDtypeStruct((nf, EMBED, BATCH), jnp.float32),
    )


def _make_gather(nf, off):
    @functools.partial(
        pl.kernel,
        out_type=jax.ShapeDtypeStruct((nf, BATCH, EMBED), jnp.float32),
        mesh=plsc.VectorSubcoreMesh(core_axis_name="c", subcore_axis_name="s"),
        scratch_types=[
            pltpu.VMEM((nf, _BPW), jnp.int32),
            pltpu.VMEM((_BPW,), jnp.int32),
            pltpu.VMEM((_BPW, EMBED), jnp.float32),
            pltpu.SemaphoreType.DMA,
        ],
        compiler_params=pltpu.CompilerParams(use_tc_tiling_on_sc=False),
    )
    def gather_half(x_hbm, w_hbm, out_hbm, idx_all, idx_v, rows_v, sem):
        wid = lax.axis_index("s") * _NC + lax.axis_index("c")
        base = wid * _BPW

        pltpu.sync_copy(x_hbm.at[pl.ds(off, nf), pl.ds(base, _BPW)], idx_all)

        def field_body(i, carry):
            def remap(s, c2):
                v = idx_all[i, pl.ds(s * 16, 16)]
                r = ((v >> 9) << 9) + ((v & 127) << 2) + ((v >> 7) & 3)
                idx_v[pl.ds(s * 16, 16)] = r + i * _VPAD
                return c2

            lax.fori_loop(0, _BPW // 16, remap, 0)
            pltpu.async_copy(w_hbm.at[idx_v], rows_v, sem).wait()
            pltpu.sync_copy(rows_v, out_hbm.at[i, pl.ds(base, _BPW)])
            return carry

        lax.fori_loop(0, nf, field_body, 0)

    return gather_half


_repack1 = _make_repack(N_FIELDS, 0)
_gather1 = _make_gather(N_FIELDS, 0)
_unpack1 = _make_unpack(N_FIELDS)


def _half(repack, gather, unpack, nf, wt, xt):
    w_pk = repack(wt).reshape(nf * _VPAD, EMBED)
    out = gather(xt, w_pk)
    y = out.reshape(nf, BATCH * EMBED // 128, 128)
    out_bm = unpack(y)
    return jnp.transpose(out_bm, (0, 2, 1))


def kernel(x, W):
    wt = jnp.transpose(W, (0, 2, 1))
    xt = jnp.transpose(x.astype(jnp.int32))
    outs1 = _half(_repack1, _gather1, _unpack1, N_FIELDS, wt, xt)
    return tuple(outs1[i] for i in range(N_FIELDS))


# speedup vs baseline: 1.0932x; 1.0932x over previous
import functools

import jax
import jax.numpy as jnp
from jax import lax
from jax.experimental import pallas as pl
from jax.experimental.pallas import tpu as pltpu
from jax.experimental.pallas import tpu_sc as plsc

N_FIELDS = 26
VOCAB = 100000
EMBED = 32
BATCH = 16384

_info = plsc.get_sparse_core_info()
_NC, _NS = _info.num_cores, _info.num_subcores
_NW = _NC * _NS
_BPW = BATCH // _NW

_VBLK = 51200
_VPAD = ((VOCAB + 1 + _VBLK - 1) // _VBLK) * _VBLK

_NF1 = 17
_NF2 = N_FIELDS - _NF1


def _repack_block(x_ref, o_ref):
    eye = jnp.eye(128, dtype=jnp.float32)
    x = x_ref[0]
    for s in range(_VBLK // 512):
        pieces = [
            jax.lax.dot_general(
                eye,
                x[:, s * 512 + d * 128:s * 512 + (d + 1) * 128],
                (((1,), (1,)), ((), ())),
                preferred_element_type=jnp.float32,
            )
            for d in range(4)
        ]
        o_ref[0, pl.ds(s * 128, 128)] = jnp.concatenate(pieces, axis=1)


def _make_repack(nf, off):
    return pl.pallas_call(
        _repack_block,
        grid=(nf, _VPAD // _VBLK),
        in_specs=[pl.BlockSpec((1, EMBED, _VBLK), lambda i, j: (i + off, 0, j))],
        out_specs=pl.BlockSpec((1, _VBLK // 4, 4 * EMBED), lambda i, j: (i, j, 0)),
        out_shape=jax.ShapeDtypeStruct((nf, _VPAD // 4, 4 * EMBED), jnp.float32),
    )


_UROWS = 4096
_UB = _UROWS * 4


def _unpack_block(y_ref, o_ref):
    r_io = lax.broadcasted_iota(jnp.int32, (128, 512), 0)
    c_io = lax.broadcasted_iota(jnp.int32, (128, 512), 1)
    sels = [(c_io == 4 * r_io + d).astype(jnp.float32) for d in range(4)]
    segs = []
    for g in range(_UROWS // 128):
        acc = None
        for d in range(4):
            piece = jax.lax.dot_general(
                y_ref[0, pl.ds(g * 128, 128), pl.ds(d * EMBED, EMBED)],
                sels[d],
                (((0,), (0,)), ((), ())),
                preferred_element_type=jnp.float32,
            )
            acc = piece if acc is None else acc + piece
        segs.append(acc)
    o_ref[0] = jnp.concatenate(segs, axis=1)


def _make_unpack(nf):
    return pl.pallas_call(
        _unpack_block,
        grid=(nf, BATCH // _UB),
        in_specs=[pl.BlockSpec((1, _UROWS, 128), lambda i, j: (i, j, 0))],
        out_specs=pl.BlockSpec((1, EMBED, _UB), lambda i, j: (i, 0, j)),
        out_shape=jax.ShapeDtypeStruct((nf, EMBED, BATCH), jnp.float32),
    )


def _make_gather(nf, off):
    @functools.partial(
        pl.kernel,
        out_type=jax.ShapeDtypeStruct((nf, BATCH, EMBED), jnp.float32),
        mesh=plsc.VectorSubcoreMesh(core_axis_name="c", subcore_axis_name="s"),
        scratch_types=[
            pltpu.VMEM((nf, _BPW), jnp.int32),
            pltpu.VMEM((_BPW,), jnp.int32),
            pltpu.VMEM((_BPW, EMBED), jnp.float32),
            pltpu.SemaphoreType.DMA,
        ],
        compiler_params=pltpu.CompilerParams(use_tc_tiling_on_sc=False),
    )
    def gather_half(x_hbm, w_hbm, out_hbm, idx_all, idx_v, rows_v, sem):
        wid = lax.axis_index("s") * _NC + lax.axis_index("c")
        base = wid * _BPW

        pltpu.sync_copy(x_hbm.at[pl.ds(off, nf), pl.ds(base, _BPW)], idx_all)

        def field_body(i, carry):
            def remap(s, c2):
                v = idx_all[i, pl.ds(s * 16, 16)]
                r = ((v >> 9) << 9) + ((v & 127) << 2) + ((v >> 7) & 3)
                idx_v[pl.ds(s * 16, 16)] = r + i * _VPAD
                return c2

            lax.fori_loop(0, _BPW // 16, remap, 0)
            pltpu.async_copy(w_hbm.at[idx_v], rows_v, sem).wait()
            pltpu.sync_copy(rows_v, out_hbm.at[i, pl.ds(base, _BPW)])
            return carry

        lax.fori_loop(0, nf, field_body, 0)

    return gather_half


_repack1 = _make_repack(_NF1, 0)
_repack2 = _make_repack(_NF2, _NF1)
_gather1 = _make_gather(_NF1, 0)
_gather2 = _make_gather(_NF2, _NF1)
_unpack1 = _make_unpack(_NF1)
_unpack2 = _make_unpack(_NF2)


def _half(repack, gather, unpack, nf, wt, xt):
    w_pk = repack(wt).reshape(nf * _VPAD, EMBED)
    out = gather(xt, w_pk)
    y = out.reshape(nf, BATCH * EMBED // 128, 128)
    out_bm = unpack(y)
    return jnp.transpose(out_bm, (0, 2, 1))


def kernel(x, W):
    wt = jnp.transpose(W, (0, 2, 1))
    xt = jnp.transpose(x.astype(jnp.int32))
    outs1 = _half(_repack1, _gather1, _unpack1, _NF1, wt, xt)
    outs2 = _half(_repack2, _gather2, _unpack2, _NF2, wt, xt)
    return tuple(outs1[i] for i in range(_NF1)) + tuple(
        outs2[i] for i in range(_NF2)
    )


# speedup vs baseline: 1.0973x; 1.0037x over previous
import functools

import jax
import jax.numpy as jnp
from jax import lax
from jax.experimental import pallas as pl
from jax.experimental.pallas import tpu as pltpu
from jax.experimental.pallas import tpu_sc as plsc

N_FIELDS = 26
VOCAB = 100000
EMBED = 32
BATCH = 16384

_info = plsc.get_sparse_core_info()
_NC, _NS = _info.num_cores, _info.num_subcores
_NW = _NC * _NS
_BPW = BATCH // _NW

_VBLK = 51200
_VPAD = ((VOCAB + 1 + _VBLK - 1) // _VBLK) * _VBLK

_NF1 = N_FIELDS // 2
_NF2 = N_FIELDS - _NF1


def _repack_block(x_ref, o_ref):
    eye = jnp.eye(128, dtype=jnp.float32)
    x = x_ref[0]
    for s in range(_VBLK // 512):
        pieces = [
            jax.lax.dot_general(
                eye,
                x[:, s * 512 + d * 128:s * 512 + (d + 1) * 128],
                (((1,), (1,)), ((), ())),
                preferred_element_type=jnp.float32,
            )
            for d in range(4)
        ]
        o_ref[0, pl.ds(s * 128, 128)] = jnp.concatenate(pieces, axis=1)


def _make_repack(nf, off):
    return pl.pallas_call(
        _repack_block,
        grid=(nf, _VPAD // _VBLK),
        in_specs=[pl.BlockSpec((1, EMBED, _VBLK), lambda i, j: (i + off, 0, j))],
        out_specs=pl.BlockSpec((1, _VBLK // 4, 4 * EMBED), lambda i, j: (i, j, 0)),
        out_shape=jax.ShapeDtypeStruct((nf, _VPAD // 4, 4 * EMBED), jnp.float32),
    )


_UROWS = 4096
_UB = _UROWS * 4


def _unpack_block(y_ref, o_ref):
    r_io = lax.broadcasted_iota(jnp.int32, (128, 512), 0)
    c_io = lax.broadcasted_iota(jnp.int32, (128, 512), 1)
    sels = [(c_io == 4 * r_io + d).astype(jnp.float32) for d in range(4)]
    segs = []
    for g in range(_UROWS // 128):
        acc = None
        for d in range(4):
            piece = jax.lax.dot_general(
                y_ref[0, pl.ds(g * 128, 128), pl.ds(d * EMBED, EMBED)],
                sels[d],
                (((0,), (0,)), ((), ())),
                preferred_element_type=jnp.float32,
            )
            acc = piece if acc is None else acc + piece
        segs.append(acc)
    o_ref[0] = jnp.concatenate(segs, axis=1)


def _make_unpack(nf):
    return pl.pallas_call(
        _unpack_block,
        grid=(nf, BATCH // _UB),
        in_specs=[pl.BlockSpec((1, _UROWS, 128), lambda i, j: (i, j, 0))],
        out_specs=pl.BlockSpec((1, EMBED, _UB), lambda i, j: (i, 0, j)),
        out_shape=jax.ShapeDtypeStruct((nf, EMBED, BATCH), jnp.float32),
    )


def _make_gather(nf, off):
    @functools.partial(
        pl.kernel,
        out_type=jax.ShapeDtypeStruct((nf, BATCH, EMBED), jnp.float32),
        mesh=plsc.VectorSubcoreMesh(core_axis_name="c", subcore_axis_name="s"),
        scratch_types=[
            pltpu.VMEM((nf, _BPW), jnp.int32),
            pltpu.VMEM((_BPW,), jnp.int32),
            pltpu.VMEM((_BPW, EMBED), jnp.float32),
            pltpu.SemaphoreType.DMA,
        ],
        compiler_params=pltpu.CompilerParams(use_tc_tiling_on_sc=False),
    )
    def gather_half(x_hbm, w_hbm, out_hbm, idx_all, idx_v, rows_v, sem):
        wid = lax.axis_index("s") * _NC + lax.axis_index("c")
        base = wid * _BPW

        pltpu.sync_copy(x_hbm.at[pl.ds(off, nf), pl.ds(base, _BPW)], idx_all)

        def field_body(i, carry):
            def remap(s, c2):
                v = idx_all[i, pl.ds(s * 16, 16)]
                r = ((v >> 9) << 9) + ((v & 127) << 2) + ((v >> 7) & 3)
                idx_v[pl.ds(s * 16, 16)] = r + i * _VPAD
                return c2

            lax.fori_loop(0, _BPW // 16, remap, 0)
            pltpu.async_copy(w_hbm.at[idx_v], rows_v, sem).wait()
            pltpu.sync_copy(rows_v, out_hbm.at[i, pl.ds(base, _BPW)])
            return carry

        lax.fori_loop(0, nf, field_body, 0)

    return gather_half


_repack1 = _make_repack(_NF1, 0)
_repack2 = _make_repack(_NF2, _NF1)
_gather1 = _make_gather(_NF1, 0)
_gather2 = _make_gather(_NF2, _NF1)
_unpack1 = _make_unpack(_NF1)
_unpack2 = _make_unpack(_NF2)


def _half(repack, gather, unpack, nf, wt, xt):
    w_pk = repack(wt).reshape(nf * _VPAD, EMBED)
    out = gather(xt, w_pk)
    y = out.reshape(nf, BATCH * EMBED // 128, 128)
    out_bm = unpack(y)
    return jnp.transpose(out_bm, (0, 2, 1))


def kernel(x, W):
    wt = jnp.transpose(W, (0, 2, 1))
    xt = jnp.transpose(x.astype(jnp.int32))
    outs1 = _half(_repack1, _gather1, _unpack1, _NF1, wt, xt)
    outs2 = _half(_repack2, _gather2, _unpack2, _NF2, wt, xt)
    return tuple(outs1[i] for i in range(_NF1)) + tuple(
        outs2[i] for i in range(_NF2)
    )
